# baseline (speedup 1.0000x reference)
import dataclasses
import functools

import jax
import jax.numpy as jnp
from jax import lax
from jax.experimental import pallas as pl
from jax.experimental.pallas import tpu as pltpu
from jax.experimental.pallas import tpu_sc as plsc

NC = 2
NS = 16
NW = NC * NS
L = 16
K = 64
CHUNK = 128
BN = 2048


def _transpose_body(in_ref, o_ref):
    o_ref[...] = in_ref[...].T


def _tc_transpose(t):
    kd, n = t.shape
    return pl.pallas_call(
        _transpose_body,
        grid=(pl.cdiv(n, BN),),
        in_specs=[pl.BlockSpec((kd, BN), lambda i: (0, i))],
        out_specs=pl.BlockSpec((BN, kd), lambda i: (i, 0)),
        out_shape=jax.ShapeDtypeStruct((n, kd), jnp.float32),
        compiler_params=pltpu.CompilerParams(
            dimension_semantics=("parallel",)),
    )(t)


def _sc_mf_kernel(B):
    b_per_w = B // NW
    n_chunks = b_per_w // CHUNK
    mesh = plsc.VectorSubcoreMesh(core_axis_name="c", subcore_axis_name="s")
    cp = pltpu.CompilerParams()
    if "needs_layout_passes" in pltpu.CompilerParams.__dataclass_fields__:
        cp = dataclasses.replace(cp, needs_layout_passes=False)
    if "use_tc_tiling_on_sc" in pltpu.CompilerParams.__dataclass_fields__:
        cp = dataclasses.replace(cp, use_tc_tiling_on_sc=True)

    @functools.partial(
        pl.kernel,
        out_type=jax.ShapeDtypeStruct((NW, b_per_w), jnp.float32),
        mesh=mesh,
        compiler_params=cp,
        scratch_types=[
            pltpu.VMEM((b_per_w,), jnp.int32),
            pltpu.VMEM((b_per_w,), jnp.int32),
            pltpu.VMEM((CHUNK, K), jnp.float32),
            pltpu.VMEM((CHUNK, K), jnp.float32),
            pltpu.VMEM((CHUNK, K), jnp.float32),
            pltpu.VMEM((CHUNK, K), jnp.float32),
            pltpu.VMEM((b_per_w,), jnp.float32),
            pltpu.SemaphoreType.DMA,
            pltpu.SemaphoreType.DMA,
        ],
    )
    def kern(u_idx_hbm, i_idx_hbm, uf_hbm, if_hbm, out_hbm,
             u_idxv, i_idxv, u_a, u_b, v_a, v_b, out_v, sem_a, sem_b):
        wid = lax.axis_index("s") * NC + lax.axis_index("c")

        pltpu.sync_copy(u_idx_hbm.at[wid], u_idxv)
        pltpu.sync_copy(i_idx_hbm.at[wid], i_idxv)

        u_bufs = [u_a, u_b]
        v_bufs = [v_a, v_b]
        sems = [sem_a, sem_b]

        def fire(c):
            u_buf, v_buf, s = u_bufs[c % 2], v_bufs[c % 2], sems[c % 2]

            @pl.loop(0, CHUNK, step=L)
            def _(r0):
                uvec = u_idxv[pl.ds(c * CHUNK + r0, L)]
                ivec = i_idxv[pl.ds(c * CHUNK + r0, L)]
                for j in range(L):
                    pltpu.async_copy(uf_hbm.at[uvec[j]], u_buf.at[r0 + j], s)
                    pltpu.async_copy(if_hbm.at[ivec[j]], v_buf.at[r0 + j], s)

        def drain(c):
            u_buf, v_buf, s = u_bufs[c % 2], v_bufs[c % 2], sems[c % 2]
            pltpu.make_async_copy(uf_hbm.at[pl.ds(0, CHUNK)], u_buf, s).wait()
            pltpu.make_async_copy(if_hbm.at[pl.ds(0, CHUNK)], v_buf, s).wait()

        lane = lax.iota(jnp.int32, L)
        fire(0)
        for c in range(n_chunks):
            if c + 1 < n_chunks:
                fire(c + 1)
            drain(c)
            u_buf = u_bufs[c % 2]
            v_buf = v_bufs[c % 2]

            @pl.loop(0, CHUNK, step=L)
            def _(r0, c=c, u_buf=u_buf, v_buf=v_buf):
                acc = jnp.zeros((L,), jnp.float32)
                for j in range(L):
                    r = r0 + j
                    s = (u_buf[r, pl.ds(0, L)] * v_buf[r, pl.ds(0, L)]
                         + u_buf[r, pl.ds(L, L)] * v_buf[r, pl.ds(L, L)]
                         + u_buf[r, pl.ds(2 * L, L)] * v_buf[r, pl.ds(2 * L, L)]
                         + u_buf[r, pl.ds(3 * L, L)] * v_buf[r, pl.ds(3 * L, L)])
                    acc = jnp.where(lane == j, jnp.sum(s), acc)
                out_v[pl.ds(c * CHUNK + r0, L)] = acc

        pltpu.sync_copy(out_v, out_hbm.at[wid])

    return kern


def kernel(user, item, user_factors, item_factors):
    B = user.shape[0]
    b_per_w = B // NW
    u_idx = user.astype(jnp.int32).reshape(NW, b_per_w)
    i_idx = item.astype(jnp.int32).reshape(NW, b_per_w)
    uf = _tc_transpose(user_factors.T)
    if_ = _tc_transpose(item_factors.T)
    out = _sc_mf_kernel(B)(u_idx, i_idx, uf, if_)
    return out.reshape(B)


# speedup vs baseline: 1.1777x; 1.1777x over previous
import dataclasses
import functools

import jax
import jax.numpy as jnp
from jax import lax
from jax.experimental import pallas as pl
from jax.experimental.pallas import tpu as pltpu
from jax.experimental.pallas import tpu_sc as plsc

NC = 2
NS = 16
NW = NC * NS
L = 16
K = 64
CHUNK = 128
BN = 4096


def _transpose_body(in_ref, o_ref):
    xt = in_ref[...].T
    o_ref[...] = jnp.concatenate([xt, xt], axis=1)


def _tc_transpose(t):
    kd, n = t.shape
    return pl.pallas_call(
        _transpose_body,
        grid=(pl.cdiv(n, BN),),
        in_specs=[pl.BlockSpec((kd, BN), lambda i: (0, i))],
        out_specs=pl.BlockSpec((BN, 2 * kd), lambda i: (i, 0)),
        out_shape=jax.ShapeDtypeStruct((n, 2 * kd), jnp.float32),
        compiler_params=pltpu.CompilerParams(
            dimension_semantics=("arbitrary",)),
    )(t)


def _sc_mf_kernel(B):
    b_per_w = B // NW
    n_chunks = b_per_w // CHUNK
    mesh = plsc.VectorSubcoreMesh(core_axis_name="c", subcore_axis_name="s")
    cp = pltpu.CompilerParams()
    if "needs_layout_passes" in pltpu.CompilerParams.__dataclass_fields__:
        cp = dataclasses.replace(cp, needs_layout_passes=False)

    @functools.partial(
        pl.kernel,
        out_type=jax.ShapeDtypeStruct((NW, b_per_w), jnp.float32),
        mesh=mesh,
        compiler_params=cp,
        scratch_types=[
            pltpu.VMEM((n_chunks, CHUNK), jnp.int32),
            pltpu.VMEM((n_chunks, CHUNK), jnp.int32),
            pltpu.VMEM((CHUNK, 2 * K), jnp.float32),
            pltpu.VMEM((CHUNK, 2 * K), jnp.float32),
            pltpu.VMEM((CHUNK, 2 * K), jnp.float32),
            pltpu.VMEM((CHUNK, 2 * K), jnp.float32),
            pltpu.VMEM((b_per_w,), jnp.float32),
            pltpu.SemaphoreType.DMA,
            pltpu.SemaphoreType.DMA,
        ],
    )
    def kern(u_idx_hbm, i_idx_hbm, uf_hbm, if_hbm, out_hbm,
             u_idx, i_idx, u_a, u_b, v_a, v_b, out_v, sem_a, sem_b):
        wid = lax.axis_index("s") * NC + lax.axis_index("c")

        pltpu.sync_copy(u_idx_hbm.at[wid], u_idx)
        pltpu.sync_copy(i_idx_hbm.at[wid], i_idx)

        u_bufs = [u_a, u_b]
        v_bufs = [v_a, v_b]
        sems = [sem_a, sem_b]

        def fire(c):
            s = sems[c % 2]
            return [
                pltpu.async_copy(uf_hbm.at[u_idx.at[c]], u_bufs[c % 2], s),
                pltpu.async_copy(if_hbm.at[i_idx.at[c]], v_bufs[c % 2], s),
            ]

        lane = lax.iota(jnp.int32, L)
        pending = {0: fire(0)}
        for c in range(n_chunks):
            if c + 1 < n_chunks:
                pending[c + 1] = fire(c + 1)
            for cp_ in pending.pop(c):
                cp_.wait()
            u_buf = u_bufs[c % 2]
            v_buf = v_bufs[c % 2]

            @pl.loop(0, CHUNK, step=L)
            def _(r0, c=c, u_buf=u_buf, v_buf=v_buf):
                acc = jnp.zeros((L,), jnp.float32)
                for j in range(L):
                    r = r0 + j
                    s = (u_buf[r, pl.ds(0, L)] * v_buf[r, pl.ds(0, L)]
                         + u_buf[r, pl.ds(L, L)] * v_buf[r, pl.ds(L, L)]
                         + u_buf[r, pl.ds(2 * L, L)] * v_buf[r, pl.ds(2 * L, L)]
                         + u_buf[r, pl.ds(3 * L, L)] * v_buf[r, pl.ds(3 * L, L)])
                    acc = jnp.where(lane == j, jnp.sum(s), acc)
                out_v[pl.ds(c * CHUNK + r0, L)] = acc

        pltpu.sync_copy(out_v, out_hbm.at[wid])

    return kern


def kernel(user, item, user_factors, item_factors):
    B = user.shape[0]
    b_per_w = B // NW
    n_chunks = b_per_w // CHUNK
    uf2 = _tc_transpose(user_factors.T)
    if2 = _tc_transpose(item_factors.T)
    u_idx = user.astype(jnp.int32).reshape(NW, n_chunks, CHUNK)
    i_idx = item.astype(jnp.int32).reshape(NW, n_chunks, CHUNK)
    out = _sc_mf_kernel(B)(u_idx, i_idx, uf2, if2)
    return out.reshape(B)


# speedup vs baseline: 1.4079x; 1.1954x over previous
import dataclasses
import functools

import jax
import jax.numpy as jnp
from jax import lax
from jax.experimental import pallas as pl
from jax.experimental.pallas import tpu as pltpu
from jax.experimental.pallas import tpu_sc as plsc

NC = 2
NS = 16
NW = NC * NS
L = 16
K = 64
CHUNK = 128
BN = 8192


def _transpose_body(in_ref, o_ref):
    xt = in_ref[...].T
    o_ref[...] = jnp.concatenate([xt, xt], axis=1)


def _tc_transpose(t):
    kd, n = t.shape
    return pl.pallas_call(
        _transpose_body,
        grid=(pl.cdiv(n, BN),),
        in_specs=[pl.BlockSpec((kd, BN), lambda i: (0, i))],
        out_specs=pl.BlockSpec((BN, 2 * kd), lambda i: (i, 0)),
        out_shape=jax.ShapeDtypeStruct((n, 2 * kd), jnp.float32),
        compiler_params=pltpu.CompilerParams(
            dimension_semantics=("arbitrary",)),
    )(t)


def _sc_mf_kernel(B):
    b_per_w = B // NW
    n_chunks = b_per_w // CHUNK
    mesh = plsc.VectorSubcoreMesh(core_axis_name="c", subcore_axis_name="s")
    cp = pltpu.CompilerParams()
    if "needs_layout_passes" in pltpu.CompilerParams.__dataclass_fields__:
        cp = dataclasses.replace(cp, needs_layout_passes=False)

    @functools.partial(
        pl.kernel,
        out_type=jax.ShapeDtypeStruct((NW, b_per_w), jnp.float32),
        mesh=mesh,
        compiler_params=cp,
        scratch_types=[
            pltpu.VMEM((n_chunks, CHUNK), jnp.int32),
            pltpu.VMEM((n_chunks, CHUNK), jnp.int32),
            pltpu.VMEM((CHUNK, 2 * K), jnp.float32),
            pltpu.VMEM((CHUNK, 2 * K), jnp.float32),
            pltpu.VMEM((CHUNK, 2 * K), jnp.float32),
            pltpu.VMEM((CHUNK, 2 * K), jnp.float32),
            pltpu.VMEM((b_per_w,), jnp.float32),
            pltpu.SemaphoreType.DMA,
            pltpu.SemaphoreType.DMA,
        ],
    )
    def kern(u_idx_hbm, i_idx_hbm, uf_hbm, if_hbm, out_hbm,
             u_idx, i_idx, u_a, u_b, v_a, v_b, out_v, sem_a, sem_b):
        wid = lax.axis_index("s") * NC + lax.axis_index("c")

        pltpu.sync_copy(u_idx_hbm.at[wid], u_idx)
        pltpu.sync_copy(i_idx_hbm.at[wid], i_idx)

        u_bufs = [u_a, u_b]
        v_bufs = [v_a, v_b]
        sems = [sem_a, sem_b]

        def fire(c):
            s = sems[c % 2]
            return [
                pltpu.async_copy(uf_hbm.at[u_idx.at[c]], u_bufs[c % 2], s),
                pltpu.async_copy(if_hbm.at[i_idx.at[c]], v_bufs[c % 2], s),
            ]

        lane = lax.iota(jnp.int32, L)
        pending = {0: fire(0)}
        for c in range(n_chunks):
            if c + 1 < n_chunks:
                pending[c + 1] = fire(c + 1)
            for cp_ in pending.pop(c):
                cp_.wait()
            u_buf = u_bufs[c % 2]
            v_buf = v_bufs[c % 2]

            @pl.loop(0, CHUNK, step=L)
            def _(r0, c=c, u_buf=u_buf, v_buf=v_buf):
                acc = jnp.zeros((L,), jnp.float32)
                for j in range(L):
                    r = r0 + j
                    s = (u_buf[r, pl.ds(0, L)] * v_buf[r, pl.ds(0, L)]
                         + u_buf[r, pl.ds(L, L)] * v_buf[r, pl.ds(L, L)]
                         + u_buf[r, pl.ds(2 * L, L)] * v_buf[r, pl.ds(2 * L, L)]
                         + u_buf[r, pl.ds(3 * L, L)] * v_buf[r, pl.ds(3 * L, L)])
                    acc = jnp.where(lane == j, jnp.sum(s), acc)
                out_v[pl.ds(c * CHUNK + r0, L)] = acc

        pltpu.sync_copy(out_v, out_hbm.at[wid])

    return kern


def kernel(user, item, user_factors, item_factors):
    B = user.shape[0]
    b_per_w = B // NW
    n_chunks = b_per_w // CHUNK
    uf2 = _tc_transpose(user_factors.T)
    if2 = _tc_transpose(item_factors.T)
    u_idx = user.astype(jnp.int32).reshape(NW, n_chunks, CHUNK)
    i_idx = item.astype(jnp.int32).reshape(NW, n_chunks, CHUNK)
    out = _sc_mf_kernel(B)(u_idx, i_idx, uf2, if2)
    return out.reshape(B)


# speedup vs baseline: 1.6156x; 1.1476x over previous
import dataclasses
import functools

import jax
import jax.numpy as jnp
from jax import lax
from jax.experimental import pallas as pl
from jax.experimental.pallas import tpu as pltpu
from jax.experimental.pallas import tpu_sc as plsc

NC = 2
NS = 16
NW = NC * NS
L = 16
K = 64
CHUNK = 128
BN = 8192
BNU = 4096
SPLIT_BLOCKS = 123
S = SPLIT_BLOCKS * BNU


def _dup_transpose_body(in_ref, o_ref):
    xt = in_ref[...].T
    o_ref[...] = jnp.concatenate([xt, xt], axis=1)


def _tc_dup_transpose(t):
    kd, n = t.shape
    return pl.pallas_call(
        _dup_transpose_body,
        grid=(pl.cdiv(n, BN),),
        in_specs=[pl.BlockSpec((kd, BN), lambda i: (0, i))],
        out_specs=pl.BlockSpec((BN, 2 * kd), lambda i: (i, 0)),
        out_shape=jax.ShapeDtypeStruct((n, 2 * kd), jnp.float32),
        compiler_params=pltpu.CompilerParams(
            dimension_semantics=("arbitrary",)),
    )(t)


def _pack_transpose_body(lo_ref, hi_ref, o_ref):
    o_ref[...] = jnp.concatenate([lo_ref[...].T, hi_ref[...].T], axis=1)


def _tc_pack_transpose(t):
    kd, n = t.shape
    return pl.pallas_call(
        _pack_transpose_body,
        grid=(SPLIT_BLOCKS,),
        in_specs=[pl.BlockSpec((kd, BNU), lambda i: (0, i)),
                  pl.BlockSpec(
                      (kd, BNU),
                      lambda i: (0, jnp.minimum(i + SPLIT_BLOCKS,
                                                pl.cdiv(1000000, BNU) - 1)))],
        out_specs=pl.BlockSpec((BNU, 2 * kd), lambda i: (i, 0)),
        out_shape=jax.ShapeDtypeStruct((S, 2 * kd), jnp.float32),
        compiler_params=pltpu.CompilerParams(
            dimension_semantics=("arbitrary",)),
    )(t, t)


def _sc_mf_kernel(B):
    b_per_w = B // NW
    n_chunks = b_per_w // CHUNK
    mesh = plsc.VectorSubcoreMesh(core_axis_name="c", subcore_axis_name="s")
    cp = pltpu.CompilerParams()
    if "needs_layout_passes" in pltpu.CompilerParams.__dataclass_fields__:
        cp = dataclasses.replace(cp, needs_layout_passes=False)

    @functools.partial(
        pl.kernel,
        out_type=jax.ShapeDtypeStruct((NW, b_per_w), jnp.float32),
        mesh=mesh,
        compiler_params=cp,
        scratch_types=[
            pltpu.VMEM((n_chunks, CHUNK), jnp.int32),
            pltpu.VMEM((b_per_w,), jnp.int32),
            pltpu.VMEM((n_chunks, CHUNK), jnp.int32),
            pltpu.VMEM((CHUNK, 2 * K), jnp.float32),
            pltpu.VMEM((CHUNK, 2 * K), jnp.float32),
            pltpu.VMEM((CHUNK, 2 * K), jnp.float32),
            pltpu.VMEM((CHUNK, 2 * K), jnp.float32),
            pltpu.VMEM((b_per_w,), jnp.float32),
            pltpu.SemaphoreType.DMA,
            pltpu.SemaphoreType.DMA,
        ],
    )
    def kern(u_idx_hbm, u_par_hbm, i_idx_hbm, uf_hbm, if_hbm, out_hbm,
             u_idx, u_par, i_idx, u_a, u_b, v_a, v_b, out_v, sem_a, sem_b):
        wid = lax.axis_index("s") * NC + lax.axis_index("c")

        pltpu.sync_copy(u_idx_hbm.at[wid], u_idx)
        pltpu.sync_copy(u_par_hbm.at[wid], u_par)
        pltpu.sync_copy(i_idx_hbm.at[wid], i_idx)

        u_bufs = [u_a, u_b]
        v_bufs = [v_a, v_b]
        sems = [sem_a, sem_b]

        def fire(c):
            s = sems[c % 2]
            return [
                pltpu.async_copy(uf_hbm.at[u_idx.at[c]], u_bufs[c % 2], s),
                pltpu.async_copy(if_hbm.at[i_idx.at[c]], v_bufs[c % 2], s),
            ]

        lane = lax.iota(jnp.int32, L)
        pending = {0: fire(0)}
        for c in range(n_chunks):
            if c + 1 < n_chunks:
                pending[c + 1] = fire(c + 1)
            for cp_ in pending.pop(c):
                cp_.wait()
            u_buf = u_bufs[c % 2]
            v_buf = v_bufs[c % 2]

            @pl.loop(0, CHUNK, step=L)
            def _(r0, c=c, u_buf=u_buf, v_buf=v_buf):
                acc_lo = jnp.zeros((L,), jnp.float32)
                acc_hi = jnp.zeros((L,), jnp.float32)
                for j in range(L):
                    r = r0 + j
                    lo = (u_buf[r, pl.ds(0, L)] * v_buf[r, pl.ds(0, L)]
                          + u_buf[r, pl.ds(L, L)] * v_buf[r, pl.ds(L, L)]
                          + u_buf[r, pl.ds(2 * L, L)] * v_buf[r, pl.ds(2 * L, L)]
                          + u_buf[r, pl.ds(3 * L, L)] * v_buf[r, pl.ds(3 * L, L)])
                    hi = (u_buf[r, pl.ds(4 * L, L)] * v_buf[r, pl.ds(4 * L, L)]
                          + u_buf[r, pl.ds(5 * L, L)] * v_buf[r, pl.ds(5 * L, L)]
                          + u_buf[r, pl.ds(6 * L, L)] * v_buf[r, pl.ds(6 * L, L)]
                          + u_buf[r, pl.ds(7 * L, L)] * v_buf[r, pl.ds(7 * L, L)])
                    acc_lo = jnp.where(lane == j, jnp.sum(lo), acc_lo)
                    acc_hi = jnp.where(lane == j, jnp.sum(hi), acc_hi)
                par = u_par[pl.ds(c * CHUNK + r0, L)]
                out_v[pl.ds(c * CHUNK + r0, L)] = jnp.where(
                    par != 0, acc_hi, acc_lo)

        pltpu.sync_copy(out_v, out_hbm.at[wid])

    return kern


def kernel(user, item, user_factors, item_factors):
    B = user.shape[0]
    b_per_w = B // NW
    n_chunks = b_per_w // CHUNK
    uf2 = _tc_pack_transpose(user_factors.T)
    if2 = _tc_dup_transpose(item_factors.T)
    user = user.astype(jnp.int32)
    u_gather = jnp.where(user < S, user, user - S).reshape(NW, n_chunks, CHUNK)
    u_parity = (user >= S).astype(jnp.int32).reshape(NW, b_per_w)
    i_idx = item.astype(jnp.int32).reshape(NW, n_chunks, CHUNK)
    out = _sc_mf_kernel(B)(u_gather, u_parity, i_idx, uf2, if2)
    return out.reshape(B)


# speedup vs baseline: 1.7892x; 1.1074x over previous
import dataclasses
import functools

import jax
import jax.numpy as jnp
from jax import lax
from jax.experimental import pallas as pl
from jax.experimental.pallas import tpu as pltpu
from jax.experimental.pallas import tpu_sc as plsc

NC = 2
NS = 16
NW = NC * NS
L = 16
K = 64
CHUNK = 128
BN = 8192
BNU = 8192
SPLIT_BLOCKS = 62
S = SPLIT_BLOCKS * BNU


def _dup_transpose_body(in_ref, o_ref):
    xt = in_ref[...].T
    o_ref[...] = jnp.concatenate([xt, xt], axis=1)


def _tc_dup_transpose(t):
    kd, n = t.shape
    return pl.pallas_call(
        _dup_transpose_body,
        grid=(pl.cdiv(n, BN),),
        in_specs=[pl.BlockSpec((kd, BN), lambda i: (0, i))],
        out_specs=pl.BlockSpec((BN, 2 * kd), lambda i: (i, 0)),
        out_shape=jax.ShapeDtypeStruct((n, 2 * kd), jnp.float32),
        compiler_params=pltpu.CompilerParams(
            dimension_semantics=("arbitrary",)),
    )(t)


def _pack_transpose_body(lo_ref, hi_ref, o_ref):
    o_ref[...] = jnp.concatenate([lo_ref[...].T, hi_ref[...].T], axis=1)


def _tc_pack_transpose(t):
    kd, n = t.shape
    return pl.pallas_call(
        _pack_transpose_body,
        grid=(SPLIT_BLOCKS,),
        in_specs=[pl.BlockSpec((kd, BNU), lambda i: (0, i)),
                  pl.BlockSpec(
                      (kd, BNU),
                      lambda i: (0, jnp.minimum(i + SPLIT_BLOCKS,
                                                pl.cdiv(1000000, BNU) - 1)))],
        out_specs=pl.BlockSpec((BNU, 2 * kd), lambda i: (i, 0)),
        out_shape=jax.ShapeDtypeStruct((S, 2 * kd), jnp.float32),
        compiler_params=pltpu.CompilerParams(
            dimension_semantics=("arbitrary",)),
    )(t, t)


def _sc_mf_kernel(B):
    b_per_w = B // NW
    n_chunks = b_per_w // CHUNK
    mesh = plsc.VectorSubcoreMesh(core_axis_name="c", subcore_axis_name="s")
    cp = pltpu.CompilerParams()
    if "needs_layout_passes" in pltpu.CompilerParams.__dataclass_fields__:
        cp = dataclasses.replace(cp, needs_layout_passes=False)

    @functools.partial(
        pl.kernel,
        out_type=jax.ShapeDtypeStruct((NW, b_per_w), jnp.float32),
        mesh=mesh,
        compiler_params=cp,
        scratch_types=[
            pltpu.VMEM((n_chunks, CHUNK), jnp.int32),
            pltpu.VMEM((b_per_w,), jnp.int32),
            pltpu.VMEM((n_chunks, CHUNK), jnp.int32),
            pltpu.VMEM((CHUNK, 2 * K), jnp.float32),
            pltpu.VMEM((CHUNK, 2 * K), jnp.float32),
            pltpu.VMEM((CHUNK, 2 * K), jnp.float32),
            pltpu.VMEM((CHUNK, 2 * K), jnp.float32),
            pltpu.VMEM((b_per_w,), jnp.float32),
            pltpu.SemaphoreType.DMA,
            pltpu.SemaphoreType.DMA,
        ],
    )
    def kern(u_idx_hbm, u_par_hbm, i_idx_hbm, uf_hbm, if_hbm, out_hbm,
             u_idx, u_par, i_idx, u_a, u_b, v_a, v_b, out_v, sem_a, sem_b):
        wid = lax.axis_index("s") * NC + lax.axis_index("c")

        pltpu.sync_copy(u_idx_hbm.at[wid], u_idx)
        pltpu.sync_copy(u_par_hbm.at[wid], u_par)
        pltpu.sync_copy(i_idx_hbm.at[wid], i_idx)

        u_bufs = [u_a, u_b]
        v_bufs = [v_a, v_b]
        sems = [sem_a, sem_b]

        def fire(c):
            s = sems[c % 2]
            return [
                pltpu.async_copy(uf_hbm.at[u_idx.at[c]], u_bufs[c % 2], s),
                pltpu.async_copy(if_hbm.at[i_idx.at[c]], v_bufs[c % 2], s),
            ]

        lane = lax.iota(jnp.int32, L)
        pending = {0: fire(0)}
        for c in range(n_chunks):
            if c + 1 < n_chunks:
                pending[c + 1] = fire(c + 1)
            for cp_ in pending.pop(c):
                cp_.wait()
            u_buf = u_bufs[c % 2]
            v_buf = v_bufs[c % 2]

            @pl.loop(0, CHUNK, step=L)
            def _(r0, c=c, u_buf=u_buf, v_buf=v_buf):
                acc_lo = jnp.zeros((L,), jnp.float32)
                acc_hi = jnp.zeros((L,), jnp.float32)
                for j in range(L):
                    r = r0 + j
                    lo = (u_buf[r, pl.ds(0, L)] * v_buf[r, pl.ds(0, L)]
                          + u_buf[r, pl.ds(L, L)] * v_buf[r, pl.ds(L, L)]
                          + u_buf[r, pl.ds(2 * L, L)] * v_buf[r, pl.ds(2 * L, L)]
                          + u_buf[r, pl.ds(3 * L, L)] * v_buf[r, pl.ds(3 * L, L)])
                    hi = (u_buf[r, pl.ds(4 * L, L)] * v_buf[r, pl.ds(4 * L, L)]
                          + u_buf[r, pl.ds(5 * L, L)] * v_buf[r, pl.ds(5 * L, L)]
                          + u_buf[r, pl.ds(6 * L, L)] * v_buf[r, pl.ds(6 * L, L)]
                          + u_buf[r, pl.ds(7 * L, L)] * v_buf[r, pl.ds(7 * L, L)])
                    acc_lo = jnp.where(lane == j, jnp.sum(lo), acc_lo)
                    acc_hi = jnp.where(lane == j, jnp.sum(hi), acc_hi)
                par = u_par[pl.ds(c * CHUNK + r0, L)]
                out_v[pl.ds(c * CHUNK + r0, L)] = jnp.where(
                    par != 0, acc_hi, acc_lo)

        pltpu.sync_copy(out_v, out_hbm.at[wid])

    return kern


def kernel(user, item, user_factors, item_factors):
    B = user.shape[0]
    b_per_w = B // NW
    n_chunks = b_per_w // CHUNK
    uf2 = _tc_pack_transpose(user_factors.T)
    if2 = _tc_dup_transpose(item_factors.T)
    user = user.astype(jnp.int32)
    u_gather = jnp.where(user < S, user, user - S).reshape(NW, n_chunks, CHUNK)
    u_parity = (user >= S).astype(jnp.int32).reshape(NW, b_per_w)
    i_idx = item.astype(jnp.int32).reshape(NW, n_chunks, CHUNK)
    out = _sc_mf_kernel(B)(u_gather, u_parity, i_idx, uf2, if2)
    return out.reshape(B)


# speedup vs baseline: 1.8773x; 1.0492x over previous
import dataclasses
import functools

import jax
import jax.numpy as jnp
from jax import lax
from jax.experimental import pallas as pl
from jax.experimental.pallas import tpu as pltpu
from jax.experimental.pallas import tpu_sc as plsc

NC = 2
NS = 16
NW = NC * NS
L = 16
K = 64
CHUNK = 128
BN = 8192
BNU = 16384
SPLIT_BLOCKS = 31
S = SPLIT_BLOCKS * BNU


def _dup_transpose_body(in_ref, o_ref):
    xt = in_ref[...].T
    o_ref[...] = jnp.concatenate([xt, xt], axis=1)


def _tc_dup_transpose(t):
    kd, n = t.shape
    return pl.pallas_call(
        _dup_transpose_body,
        grid=(pl.cdiv(n, BN),),
        in_specs=[pl.BlockSpec((kd, BN), lambda i: (0, i))],
        out_specs=pl.BlockSpec((BN, 2 * kd), lambda i: (i, 0)),
        out_shape=jax.ShapeDtypeStruct((n, 2 * kd), jnp.float32),
        compiler_params=pltpu.CompilerParams(
            dimension_semantics=("arbitrary",)),
    )(t)


def _pack_transpose_body(lo_ref, hi_ref, o_ref):
    o_ref[...] = jnp.concatenate([lo_ref[...].T, hi_ref[...].T], axis=1)


def _tc_pack_transpose(t):
    kd, n = t.shape
    return pl.pallas_call(
        _pack_transpose_body,
        grid=(SPLIT_BLOCKS,),
        in_specs=[pl.BlockSpec((kd, BNU), lambda i: (0, i)),
                  pl.BlockSpec(
                      (kd, BNU),
                      lambda i: (0, jnp.minimum(i + SPLIT_BLOCKS,
                                                pl.cdiv(1000000, BNU) - 1)))],
        out_specs=pl.BlockSpec((BNU, 2 * kd), lambda i: (i, 0)),
        out_shape=jax.ShapeDtypeStruct((S, 2 * kd), jnp.float32),
        compiler_params=pltpu.CompilerParams(
            dimension_semantics=("arbitrary",)),
    )(t, t)


def _sc_mf_kernel(B):
    b_per_w = B // NW
    n_chunks = b_per_w // CHUNK
    mesh = plsc.VectorSubcoreMesh(core_axis_name="c", subcore_axis_name="s")
    cp = pltpu.CompilerParams()
    if "needs_layout_passes" in pltpu.CompilerParams.__dataclass_fields__:
        cp = dataclasses.replace(cp, needs_layout_passes=False)

    @functools.partial(
        pl.kernel,
        out_type=jax.ShapeDtypeStruct((NW, b_per_w), jnp.float32),
        mesh=mesh,
        compiler_params=cp,
        scratch_types=[
            pltpu.VMEM((n_chunks, CHUNK), jnp.int32),
            pltpu.VMEM((b_per_w,), jnp.int32),
            pltpu.VMEM((n_chunks, CHUNK), jnp.int32),
            pltpu.VMEM((CHUNK, 2 * K), jnp.float32),
            pltpu.VMEM((CHUNK, 2 * K), jnp.float32),
            pltpu.VMEM((CHUNK, 2 * K), jnp.float32),
            pltpu.VMEM((CHUNK, 2 * K), jnp.float32),
            pltpu.VMEM((b_per_w,), jnp.float32),
            pltpu.SemaphoreType.DMA,
            pltpu.SemaphoreType.DMA,
        ],
    )
    def kern(u_idx_hbm, u_par_hbm, i_idx_hbm, uf_hbm, if_hbm, out_hbm,
             u_idx, u_par, i_idx, u_a, u_b, v_a, v_b, out_v, sem_a, sem_b):
        wid = lax.axis_index("s") * NC + lax.axis_index("c")

        pltpu.sync_copy(u_idx_hbm.at[wid], u_idx)
        pltpu.sync_copy(u_par_hbm.at[wid], u_par)
        pltpu.sync_copy(i_idx_hbm.at[wid], i_idx)

        u_bufs = [u_a, u_b]
        v_bufs = [v_a, v_b]
        sems = [sem_a, sem_b]

        def fire(c):
            s = sems[c % 2]
            return [
                pltpu.async_copy(uf_hbm.at[u_idx.at[c]], u_bufs[c % 2], s),
                pltpu.async_copy(if_hbm.at[i_idx.at[c]], v_bufs[c % 2], s),
            ]

        lane = lax.iota(jnp.int32, L)
        pending = {0: fire(0)}
        for c in range(n_chunks):
            if c + 1 < n_chunks:
                pending[c + 1] = fire(c + 1)
            for cp_ in pending.pop(c):
                cp_.wait()
            u_buf = u_bufs[c % 2]
            v_buf = v_bufs[c % 2]

            @pl.loop(0, CHUNK, step=L)
            def _(r0, c=c, u_buf=u_buf, v_buf=v_buf):
                acc_lo = jnp.zeros((L,), jnp.float32)
                acc_hi = jnp.zeros((L,), jnp.float32)
                for j in range(L):
                    r = r0 + j
                    lo = (u_buf[r, pl.ds(0, L)] * v_buf[r, pl.ds(0, L)]
                          + u_buf[r, pl.ds(L, L)] * v_buf[r, pl.ds(L, L)]
                          + u_buf[r, pl.ds(2 * L, L)] * v_buf[r, pl.ds(2 * L, L)]
                          + u_buf[r, pl.ds(3 * L, L)] * v_buf[r, pl.ds(3 * L, L)])
                    hi = (u_buf[r, pl.ds(4 * L, L)] * v_buf[r, pl.ds(4 * L, L)]
                          + u_buf[r, pl.ds(5 * L, L)] * v_buf[r, pl.ds(5 * L, L)]
                          + u_buf[r, pl.ds(6 * L, L)] * v_buf[r, pl.ds(6 * L, L)]
                          + u_buf[r, pl.ds(7 * L, L)] * v_buf[r, pl.ds(7 * L, L)])
                    acc_lo = jnp.where(lane == j, jnp.sum(lo), acc_lo)
                    acc_hi = jnp.where(lane == j, jnp.sum(hi), acc_hi)
                par = u_par[pl.ds(c * CHUNK + r0, L)]
                out_v[pl.ds(c * CHUNK + r0, L)] = jnp.where(
                    par != 0, acc_hi, acc_lo)

        pltpu.sync_copy(out_v, out_hbm.at[wid])

    return kern


def kernel(user, item, user_factors, item_factors):
    B = user.shape[0]
    b_per_w = B // NW
    n_chunks = b_per_w // CHUNK
    uf2 = _tc_pack_transpose(user_factors.T)
    if2 = _tc_dup_transpose(item_factors.T)
    user = user.astype(jnp.int32)
    u_gather = jnp.where(user < S, user, user - S).reshape(NW, n_chunks, CHUNK)
    u_parity = (user >= S).astype(jnp.int32).reshape(NW, b_per_w)
    i_idx = item.astype(jnp.int32).reshape(NW, n_chunks, CHUNK)
    out = _sc_mf_kernel(B)(u_gather, u_parity, i_idx, uf2, if2)
    return out.reshape(B)
